# Initial kernel scaffold; baseline (speedup 1.0000x reference)
import jax
import jax.numpy as jnp
from jax.experimental import pallas as pl


def kernel(x, W1, b1, g1, be1, We, bee, W2, b2, g2, be2):
    raise NotImplementedError("write your pallas kernel here")


# speedup vs baseline: 4.7587x; 4.7587x over previous
import jax
import jax.numpy as jnp
from jax.experimental import pallas as pl
from jax.experimental.pallas import tpu as pltpu

_B, _C, _H, _W = 16, 96, 32, 32
_N = _H * _W
_R = 4
_M = (_H // _R) * (_W // _R)
_K = 9
_C2 = 2 * _C
_EPS_BN = 1e-5
_EPS_NORM = 1e-12
_CH = 128
_NEG = -1e30


def _dot(a, b, dims):
    return jax.lax.dot_general(a, b, (dims, ((), ())),
                               preferred_element_type=jnp.float32)


def _grapher_kernel(x_ref, W1_ref, b1_ref, g1_ref, be1_ref,
                    WeL_ref, WeR_ref, bee_ref, W2_ref, b2_ref,
                    g2_ref, be2_ref, out_ref, h_ref):
    f32 = jnp.float32

    n_i = jax.lax.broadcasted_iota(jnp.int32, (_N, _M), 0)
    m_i = jax.lax.broadcasted_iota(jnp.int32, (_N, _M), 1)
    cell = (n_i // (_W * _R)) * (_W // _R) + (n_i % _W) // _R
    P = jnp.where(cell == m_i, 1.0 / (_R * _R), 0.0).astype(f32)

    W1 = W1_ref[...]
    b1 = b1_ref[...]

    def body_a(b, carry):
        s1, s2 = carry
        z = _dot(W1, x_ref[b], ((1,), (0,))) + b1
        h_ref[b] = z
        return (s1 + jnp.sum(z, axis=1, keepdims=True),
                s2 + jnp.sum(z * z, axis=1, keepdims=True))

    zc = jnp.zeros((_C, 1), f32)
    s1, s2 = jax.lax.fori_loop(0, _B, body_a, (zc, zc))
    inv = 1.0 / (_B * _N)
    mean1 = s1 * inv
    var1 = s2 * inv - mean1 * mean1
    sc1 = g1_ref[...] * jax.lax.rsqrt(var1 + _EPS_BN)
    sh1 = be1_ref[...] - mean1 * sc1

    Wd = WeL_ref[...] - WeR_ref[...]
    WeR = WeR_ref[...]
    bee = bee_ref[...]
    W2 = W2_ref[...]
    b2 = b2_ref[...]

    def body_b(b, carry):
        t1, t2 = carry
        h = h_ref[b] * sc1 + sh1
        y = _dot(h, P, ((1,), (0,)))
        Bv = _dot(y, WeR, ((0,), (1,)))
        A = _dot(h, Wd, ((0,), (1,)))

        hn = jnp.sum(h * h, axis=0, keepdims=True)
        xn = h / jnp.maximum(jnp.sqrt(hn), _EPS_NORM)
        yn2 = jnp.sum(y * y, axis=0, keepdims=True)
        yn = y / jnp.maximum(jnp.sqrt(yn2), _EPS_NORM)
        sumy = jnp.sum(yn * yn, axis=0, keepdims=True)
        sim = _dot(xn, yn, ((0,), (0,)))
        score = 2.0 * sim - sumy

        for ci in range(_N // _CH):
            sc = jax.lax.slice_in_dim(score, ci * _CH, (ci + 1) * _CH, axis=0)
            av = sc[:, :, None]
            bv = sc[:, None, :]
            i1 = jax.lax.broadcasted_iota(jnp.int32, (_CH, _M, _M), 1)
            i2 = jax.lax.broadcasted_iota(jnp.int32, (_CH, _M, _M), 2)
            beats = jnp.where((bv > av) | ((bv == av) & (i2 < i1)), 1.0, 0.0)
            rank = jnp.sum(beats, axis=2)
            mask = rank < float(_K)
            vals = jnp.where(mask[:, :, None], Bv[None, :, :], _NEG)
            em = jnp.max(vals, axis=1)
            Ach = jax.lax.slice_in_dim(A, ci * _CH, (ci + 1) * _CH, axis=0)
            e = jnp.maximum(em + Ach + bee, 0.0)
            z2 = _dot(W2, e, ((1,), (1,))) + b2
            out_ref[b, :, ci * _CH:(ci + 1) * _CH] = z2
            t1 = t1 + jnp.sum(z2, axis=1, keepdims=True)
            t2 = t2 + jnp.sum(z2 * z2, axis=1, keepdims=True)
        return (t1, t2)

    t1, t2 = jax.lax.fori_loop(0, _B, body_b, (zc, zc))
    mean2 = t1 * inv
    var2 = t2 * inv - mean2 * mean2
    sc2 = g2_ref[...] * jax.lax.rsqrt(var2 + _EPS_BN)
    sh2 = be2_ref[...] - mean2 * sc2

    def body_c(b, _):
        out_ref[b] = out_ref[b] * sc2 + sh2 + x_ref[b]
        return 0

    jax.lax.fori_loop(0, _B, body_c, 0)


@jax.jit
def _run(x, W1, b1, g1, be1, We, bee, W2, b2, g2, be2):
    xr = x.reshape(_B, _C, _N)
    out = pl.pallas_call(
        _grapher_kernel,
        out_shape=jax.ShapeDtypeStruct((_B, _C, _N), jnp.float32),
        scratch_shapes=[pltpu.VMEM((_B, _C, _N), jnp.float32)],
    )(xr, W1, b1.reshape(_C, 1), g1.reshape(_C, 1), be1.reshape(_C, 1),
      We[:, :_C], We[:, _C:], bee.reshape(1, _C2), W2, b2.reshape(_C, 1),
      g2.reshape(_C, 1), be2.reshape(_C, 1))
    return out.reshape(_B, _C, _H, _W)


def kernel(x, W1, b1, g1, be1, We, bee, W2, b2, g2, be2):
    return _run(x, W1, b1, g1, be1, We, bee, W2, b2, g2, be2)


# speedup vs baseline: 16.8431x; 3.5394x over previous
import jax
import jax.numpy as jnp
from jax.experimental import pallas as pl
from jax.experimental.pallas import tpu as pltpu

_B, _C, _H, _W = 16, 96, 32, 32
_N = _H * _W
_BN = _B * _N
_R = 4
_M = (_H // _R) * (_W // _R)
_K = 9
_C2 = 2 * _C
_EPS_BN = 1e-5
_EPS_NORM = 1e-12
_CHR = 512
_NEG = -1e30


def _dot(a, b, dims):
    return jax.lax.dot_general(a, b, (dims, ((), ())),
                               preferred_element_type=jnp.float32)


def _grapher_kernel(x_ref, W1_ref, b1_ref, g1_ref, be1_ref,
                    WeL_ref, WeR_ref, bee_ref, W2_ref, b2_ref,
                    g2_ref, be2_ref, out_ref, h_ref):
    f32 = jnp.float32

    n_i = jax.lax.broadcasted_iota(jnp.int32, (_N, _M), 0)
    m_i = jax.lax.broadcasted_iota(jnp.int32, (_N, _M), 1)
    cell = (n_i // (_W * _R)) * (_W // _R) + (n_i % _W) // _R
    P = jnp.where(cell == m_i, 1.0 / (_R * _R), 0.0).astype(f32)

    ii = jax.lax.broadcasted_iota(jnp.int32, (_M, _M), 0)
    jj = jax.lax.broadcasted_iota(jnp.int32, (_M, _M), 1)
    eye = jnp.where(ii == jj, 1.0, 0.0).astype(f32)

    i_m = jax.lax.broadcasted_iota(jnp.int32, (_M, _M, _CHR), 0)
    i_mp = jax.lax.broadcasted_iota(jnp.int32, (_M, _M, _CHR), 1)
    tie_lt = i_mp < i_m

    X = x_ref[...]

    Z = _dot(W1_ref[...], X, ((1,), (0,))) + b1_ref[...]
    inv = 1.0 / _BN
    mean1 = jnp.sum(Z, axis=1, keepdims=True) * inv
    var1 = jnp.sum(Z * Z, axis=1, keepdims=True) * inv - mean1 * mean1
    sc1 = g1_ref[...] * jax.lax.rsqrt(var1 + _EPS_BN)
    sh1 = be1_ref[...] - mean1 * sc1
    Hm = Z * sc1 + sh1
    h_ref[...] = Hm

    Wd = WeL_ref[...] - WeR_ref[...]
    WeR = WeR_ref[...]
    bee = bee_ref[...]
    W2 = W2_ref[...]
    b2 = b2_ref[...]

    t1 = jnp.zeros((_C, 1), f32)
    t2 = jnp.zeros((_C, 1), f32)

    for b in range(_B):
        h = h_ref[:, b * _N:(b + 1) * _N]
        y = _dot(h, P, ((1,), (0,)))
        BvT = _dot(WeR, y, ((1,), (0,)))
        A = _dot(Wd, h, ((1,), (0,)))

        hn = jnp.sum(h * h, axis=0, keepdims=True)
        xn = h / jnp.maximum(jnp.sqrt(hn), _EPS_NORM)
        yn2 = jnp.sum(y * y, axis=0, keepdims=True)
        yn = y / jnp.maximum(jnp.sqrt(yn2), _EPS_NORM)
        gram = _dot(yn, yn, ((0,), (0,)))
        sumy = jnp.sum(gram * eye, axis=1, keepdims=True)
        simT = _dot(yn, xn, ((0,), (0,)))
        score = 2.0 * simT - sumy

        ranks = []
        for c0 in range(0, _N, _CHR):
            sct = jax.lax.slice_in_dim(score, c0, c0 + _CHR, axis=1)
            a_v = sct[:, None, :]
            b_v = sct[None, :, :]
            beats = (b_v > a_v) | ((b_v == a_v) & tie_lt)
            ranks.append(jnp.sum(beats.astype(f32), axis=1))
        rank = jnp.concatenate(ranks, axis=1)

        em = None
        for k in range(_K):
            ohk = jnp.where(rank == float(k), 1.0, 0.0).astype(f32)
            selk = _dot(BvT, ohk, ((1,), (0,)))
            em = selk if em is None else jnp.maximum(em, selk)

        e = jnp.maximum(em + A + bee, 0.0)
        z2 = _dot(W2, e, ((1,), (0,))) + b2
        out_ref[:, b * _N:(b + 1) * _N] = z2
        t1 = t1 + jnp.sum(z2, axis=1, keepdims=True)
        t2 = t2 + jnp.sum(z2 * z2, axis=1, keepdims=True)

    mean2 = t1 * inv
    var2 = t2 * inv - mean2 * mean2
    sc2 = g2_ref[...] * jax.lax.rsqrt(var2 + _EPS_BN)
    sh2 = be2_ref[...] - mean2 * sc2
    out_ref[...] = out_ref[...] * sc2 + sh2 + X


@jax.jit
def _run(x, W1, b1, g1, be1, We, bee, W2, b2, g2, be2):
    xr = x.transpose(1, 0, 2, 3).reshape(_C, _BN)
    out = pl.pallas_call(
        _grapher_kernel,
        out_shape=jax.ShapeDtypeStruct((_C, _BN), jnp.float32),
        scratch_shapes=[pltpu.VMEM((_C, _BN), jnp.float32)],
    )(xr, W1, b1.reshape(_C, 1), g1.reshape(_C, 1), be1.reshape(_C, 1),
      We[:, :_C], We[:, _C:], bee.reshape(_C2, 1), W2, b2.reshape(_C, 1),
      g2.reshape(_C, 1), be2.reshape(_C, 1))
    return out.reshape(_C, _B, _N).transpose(1, 0, 2).reshape(_B, _C, _H, _W)


def kernel(x, W1, b1, g1, be1, We, bee, W2, b2, g2, be2):
    return _run(x, W1, b1, g1, be1, We, bee, W2, b2, g2, be2)
